# Initial kernel scaffold; baseline (speedup 1.0000x reference)
import jax
import jax.numpy as jnp
from jax.experimental import pallas as pl


def kernel(user_feat, item_feat, user_bias, item_bias, fc_user_W, fc_user_b, fc_item_W, fc_item_b, W1, b1, W2, b2, W3, b3, edge_users, edge_items):
    raise NotImplementedError("write your pallas kernel here")


# speedup vs baseline: 2.1890x; 2.1890x over previous
import jax
import jax.numpy as jnp
from jax import lax
from jax.experimental import pallas as pl
from jax.experimental.pallas import tpu as pltpu
from jax.experimental.pallas import tpu_sc as plsc

N_USERS = 10000
N_ITEMS = 10000
N_EDGES = 160000
D = 256
DH = 128
WN = 384

NC, NS = 2, 16


_K1_CH = 80
_K1_EPT = N_EDGES // NS
_K1_CHUNKS = _K1_EPT // _K1_CH
_K1_RPT = 632
_K1_ACC = NS * _K1_RPT


def _k1_body(ulo, uhi, ilo, ihi, eu, ei, zinit, ones_hbm,
             out_item, out_user, out_cnt,
             acc, idx_s, idx_d, rows, sem):
    cid = lax.axis_index("c")
    sid = lax.axis_index("s")
    zbase = sid * _K1_RPT

    def run_pass(tabs, dst0, dst1, out):
        pltpu.sync_copy(zinit.at[pl.ds(zbase, _K1_RPT)],
                        acc.at[pl.ds(zbase, _K1_RPT)])
        if tabs is None:
            pltpu.sync_copy(ones_hbm, rows)
        plsc.subcore_barrier()

        def chunk(k, carry):
            base = sid * _K1_EPT + k * _K1_CH
            if tabs is not None:
                tab_lo, tab_hi, src = tabs
                pltpu.sync_copy(src.at[pl.ds(base, _K1_CH)], idx_s)

            @pl.when(cid == 0)
            def _():
                pltpu.sync_copy(dst0.at[pl.ds(base, _K1_CH)], idx_d)

            @pl.when(cid == 1)
            def _():
                pltpu.sync_copy(dst1.at[pl.ds(base, _K1_CH)], idx_d)

            if tabs is not None:
                @pl.when(cid == 0)
                def _():
                    pltpu.async_copy(tab_lo.at[idx_s], rows, sem).wait()

                @pl.when(cid == 1)
                def _():
                    pltpu.async_copy(tab_hi.at[idx_s], rows, sem).wait()

            pltpu.sync_copy(rows, acc.at[idx_d], add=True)
            return carry

        lax.fori_loop(0, _K1_CHUNKS, chunk, 0)
        plsc.subcore_barrier()
        pltpu.sync_copy(acc.at[pl.ds(zbase, _K1_RPT)],
                        out.at[cid, pl.ds(zbase, _K1_RPT)])
        plsc.subcore_barrier()

    run_pass((ulo, uhi, eu), ei, ei, out_item)
    run_pass((ilo, ihi, ei), eu, eu, out_user)
    run_pass(None, ei, eu, out_cnt)


def _k1_call(ulo, uhi, ilo, ihi, eu, ei, zinit, ones_hbm):
    mesh = plsc.VectorSubcoreMesh(core_axis_name="c", subcore_axis_name="s",
                                  num_cores=NC, num_subcores=NS)
    f = pl.kernel(
        _k1_body,
        out_type=(
            jax.ShapeDtypeStruct((NC, _K1_ACC, DH), jnp.float32),
            jax.ShapeDtypeStruct((NC, _K1_ACC, DH), jnp.float32),
            jax.ShapeDtypeStruct((NC, _K1_ACC, DH), jnp.float32),
        ),
        mesh=mesh,
        scratch_types=[
            pltpu.VMEM_SHARED((_K1_ACC, DH), jnp.float32),
            pltpu.VMEM((_K1_CH,), jnp.int32),
            pltpu.VMEM((_K1_CH,), jnp.int32),
            pltpu.VMEM((_K1_CH, DH), jnp.float32),
            pltpu.SemaphoreType.DMA,
        ],
    )
    return f(ulo, uhi, ilo, ihi, eu, ei, zinit, ones_hbm)


_K2_R = 2000


def _k2_body(silo, sihi, sulo, suhi, cnti, cntu, wu, bu, wi, bi, bias,
             pu_ref, pie_ref):
    cnt_i = jnp.clip(cnti[0][:, 0:1], 1.0, None)
    h_i = jnp.concatenate([silo[0], sihi[0]], axis=1) / cnt_i
    p_i = jnp.dot(h_i, wi[...], preferred_element_type=jnp.float32) + bi[...]

    cnt_u = jnp.clip(cntu[0][:, 0:1], 1.0, None)
    h_u = jnp.concatenate([sulo[0], suhi[0]], axis=1) / cnt_u
    p_u = jnp.dot(h_u, wu[...], preferred_element_type=jnp.float32) + bu[...]

    pu_ref[...] = p_u
    pad = jnp.zeros((p_i.shape[0], WN - D - 1), jnp.float32)
    pie_ref[...] = jnp.concatenate([p_i, bias[...], pad], axis=1)


def _k2_call(sum_item, sum_user, cnt, fc_user_W, fc_user_b, fc_item_W,
             fc_item_b, item_bias):
    R = _K2_R
    grid = (N_ITEMS // R,)
    return pl.pallas_call(
        _k2_body,
        grid=grid,
        in_specs=[
            pl.BlockSpec((1, R, DH), lambda i: (0, i, 0)),
            pl.BlockSpec((1, R, DH), lambda i: (1, i, 0)),
            pl.BlockSpec((1, R, DH), lambda i: (0, i, 0)),
            pl.BlockSpec((1, R, DH), lambda i: (1, i, 0)),
            pl.BlockSpec((1, R, DH), lambda i: (0, i, 0)),
            pl.BlockSpec((1, R, DH), lambda i: (1, i, 0)),
            pl.BlockSpec((D, D), lambda i: (0, 0)),
            pl.BlockSpec((1, D), lambda i: (0, 0)),
            pl.BlockSpec((D, D), lambda i: (0, 0)),
            pl.BlockSpec((1, D), lambda i: (0, 0)),
            pl.BlockSpec((R, 1), lambda i: (i, 0)),
        ],
        out_specs=[
            pl.BlockSpec((R, D), lambda i: (i, 0)),
            pl.BlockSpec((R, WN), lambda i: (i, 0)),
        ],
        out_shape=[
            jax.ShapeDtypeStruct((N_USERS, D), jnp.float32),
            jax.ShapeDtypeStruct((N_ITEMS, WN), jnp.float32),
        ],
    )(sum_item, sum_item, sum_user, sum_user, cnt, cnt, fc_user_W,
      fc_user_b.reshape(1, D), fc_item_W, fc_item_b.reshape(1, D),
      item_bias)


_K3_EPT = N_EDGES // (NC * NS)
_K3_CH = 80
_K3_FULL = _K3_EPT // _K3_CH
_K3_TAIL = _K3_EPT - _K3_FULL * _K3_CH


def _k3_body(uf, pu, itf, pie, eu, ei,
             p_all, m_all, q_all, n_all,
             idx_u, idx_i, idx_ut, idx_it, rp, rm, rq, rn, sem):
    cid = lax.axis_index("c")
    sid = lax.axis_index("s")
    wid = sid * NC + cid
    tbase = wid * _K3_EPT

    def do_chunk(base, iu, ii, n):
        pltpu.sync_copy(eu.at[pl.ds(base, n)], iu)
        pltpu.sync_copy(ei.at[pl.ds(base, n)], ii)
        c1 = pltpu.async_copy(uf.at[iu], rp.at[pl.ds(0, n)], sem)
        c2 = pltpu.async_copy(pu.at[iu], rm.at[pl.ds(0, n)], sem)
        c3 = pltpu.async_copy(itf.at[ii], rq.at[pl.ds(0, n)], sem)
        c4 = pltpu.async_copy(pie.at[ii], rn.at[pl.ds(0, n)], sem)
        c1.wait()
        c2.wait()
        c3.wait()
        c4.wait()
        pltpu.sync_copy(rp.at[pl.ds(0, n)], p_all.at[pl.ds(base, n)])
        pltpu.sync_copy(rm.at[pl.ds(0, n)], m_all.at[pl.ds(base, n)])
        pltpu.sync_copy(rq.at[pl.ds(0, n)], q_all.at[pl.ds(base, n)])
        pltpu.sync_copy(rn.at[pl.ds(0, n)], n_all.at[pl.ds(base, n)])

    def chunk(k, carry):
        do_chunk(tbase + k * _K3_CH, idx_u, idx_i, _K3_CH)
        return carry

    lax.fori_loop(0, _K3_FULL, chunk, 0)
    do_chunk(tbase + _K3_FULL * _K3_CH, idx_ut, idx_it, _K3_TAIL)


def _k3_call(user_feat, prop_user, item_feat, prop_item_ext, eu, ei):
    mesh = plsc.VectorSubcoreMesh(core_axis_name="c", subcore_axis_name="s",
                                  num_cores=NC, num_subcores=NS)
    f = pl.kernel(
        _k3_body,
        out_type=(
            jax.ShapeDtypeStruct((N_EDGES, D), jnp.float32),
            jax.ShapeDtypeStruct((N_EDGES, D), jnp.float32),
            jax.ShapeDtypeStruct((N_EDGES, D), jnp.float32),
            jax.ShapeDtypeStruct((N_EDGES, WN), jnp.float32),
        ),
        mesh=mesh,
        scratch_types=[
            pltpu.VMEM((_K3_CH,), jnp.int32),
            pltpu.VMEM((_K3_CH,), jnp.int32),
            pltpu.VMEM((_K3_TAIL,), jnp.int32),
            pltpu.VMEM((_K3_TAIL,), jnp.int32),
            pltpu.VMEM((_K3_CH, D), jnp.float32),
            pltpu.VMEM((_K3_CH, D), jnp.float32),
            pltpu.VMEM((_K3_CH, D), jnp.float32),
            pltpu.VMEM((_K3_CH, WN), jnp.float32),
            pltpu.SemaphoreType.DMA,
        ],
    )
    return f(user_feat, prop_user, item_feat, prop_item_ext, eu, ei)


_K4_E = 1000


def _k4_body(p_ref, m_ref, q_ref, n_ref, w1, b1, w2, b2, w3, b3, out_ref):
    p = p_ref[...]
    m = m_ref[...]
    q = q_ref[...]
    ne = n_ref[...]
    n = ne[:, :D]
    bias = ne[:, D:D + 1]
    x = jnp.concatenate([p * q, p * m, n * q, n * m], axis=1)
    y = jnp.dot(x, w1[...], preferred_element_type=jnp.float32) + b1[...]
    y = jnp.maximum(y, 0.0)
    z = jnp.dot(y, w2[...], preferred_element_type=jnp.float32) + b2[...]
    z = jnp.maximum(z, 0.0)
    o = jnp.dot(z, w3[...], preferred_element_type=jnp.float32) + b3[...]
    out_ref[...] = o + bias


def _k4_call(p_all, m_all, q_all, n_all, W1, b1, W2, b2, W3, b3):
    E = _K4_E
    grid = (N_EDGES // E,)
    return pl.pallas_call(
        _k4_body,
        grid=grid,
        in_specs=[
            pl.BlockSpec((E, D), lambda i: (i, 0)),
            pl.BlockSpec((E, D), lambda i: (i, 0)),
            pl.BlockSpec((E, D), lambda i: (i, 0)),
            pl.BlockSpec((E, WN), lambda i: (i, 0)),
            pl.BlockSpec((4 * D, D), lambda i: (0, 0)),
            pl.BlockSpec((1, D), lambda i: (0, 0)),
            pl.BlockSpec((D, 64), lambda i: (0, 0)),
            pl.BlockSpec((1, 64), lambda i: (0, 0)),
            pl.BlockSpec((64, 1), lambda i: (0, 0)),
            pl.BlockSpec((1, 1), lambda i: (0, 0)),
        ],
        out_specs=pl.BlockSpec((E, 1), lambda i: (i, 0)),
        out_shape=jax.ShapeDtypeStruct((N_EDGES, 1), jnp.float32),
    )(p_all, m_all, q_all, n_all, W1, b1.reshape(1, D), W2,
      b2.reshape(1, 64), W3, b3.reshape(1, 1))


def kernel(user_feat, item_feat, user_bias, item_bias, fc_user_W, fc_user_b,
           fc_item_W, fc_item_b, W1, b1, W2, b2, W3, b3, edge_users,
           edge_items):
    f32 = jnp.float32
    ulo = user_feat[:, :DH]
    uhi = user_feat[:, DH:]
    ilo = item_feat[:, :DH]
    ihi = item_feat[:, DH:]
    zinit = jnp.zeros((_K1_ACC, DH), f32)
    ones80 = jnp.ones((_K1_CH, DH), f32)

    sum_item, sum_user, cnt = _k1_call(ulo, uhi, ilo, ihi, edge_users,
                                       edge_items, zinit, ones80)
    prop_user, prop_item_ext = _k2_call(sum_item, sum_user, cnt, fc_user_W,
                                        fc_user_b, fc_item_W, fc_item_b,
                                        item_bias)
    p_all, m_all, q_all, n_all = _k3_call(user_feat, prop_user, item_feat,
                                          prop_item_ext, edge_users,
                                          edge_items)
    return _k4_call(p_all, m_all, q_all, n_all, W1, b1, W2, b2, W3, b3)


# speedup vs baseline: 2.8731x; 1.3125x over previous
import jax
import jax.numpy as jnp
from jax import lax
from jax.experimental import pallas as pl
from jax.experimental.pallas import tpu as pltpu
from jax.experimental.pallas import tpu_sc as plsc

N_USERS = 10000
N_ITEMS = 10000
N_EDGES = 160000
D = 256
DH = 128

NC, NS = 2, 16


_K1_CH = 80
_K1_EPT = N_EDGES // NS
_K1_CPT = _K1_EPT // _K1_CH
_K1_RPT = 632
_K1_ACC = NS * _K1_RPT


def _k1_body(ulo, uhi, ilo, ihi, eu1, ei1, eu2, ei2, zinit, ones_hbm,
             out_item, out_user, out_cnt,
             acc, idxs1, idxd2, rows0, rows1, gsem, ssem0, ssem1):
    cid = lax.axis_index("c")
    sid = lax.axis_index("s")
    zbase = sid * _K1_RPT

    def drain_add(rows_b, ssem_b):
        pltpu.make_async_copy(rows_b, acc.at[idxd2.at[0]], ssem_b).wait()

    def run_pass(tabs, dst0, dst1, out):
        pltpu.sync_copy(zinit.at[pl.ds(zbase, _K1_RPT)],
                        acc.at[pl.ds(zbase, _K1_RPT)])
        if tabs is None:
            pltpu.sync_copy(ones_hbm, rows0)
        else:
            pltpu.sync_copy(tabs[2].at[pl.ds(sid * _K1_EPT, _K1_EPT)], idxs1)

        @pl.when(cid == 0)
        def _():
            pltpu.sync_copy(dst0.at[sid], idxd2)

        @pl.when(cid == 1)
        def _():
            pltpu.sync_copy(dst1.at[sid], idxd2)

        plsc.subcore_barrier()

        if tabs is None:
            def chunk(k, c):
                @pl.when(k >= 2)
                def _():
                    drain_add(rows0, ssem0)

                pltpu.async_copy(rows0, acc.at[idxd2.at[k]], ssem0, add=True)
                return c

            lax.fori_loop(0, _K1_CPT, chunk, 0)
            drain_add(rows0, ssem0)
            drain_add(rows0, ssem0)
        else:
            tab_lo, tab_hi = tabs[0], tabs[1]

            def fchunk(k2, b, rows_b, ssem_b):
                k = 2 * k2 + b

                @pl.when(k2 > 0)
                def _():
                    drain_add(rows_b, ssem_b)

                isl = idxs1.at[pl.ds(k * _K1_CH, _K1_CH)]

                @pl.when(cid == 0)
                def _():
                    pltpu.async_copy(tab_lo.at[isl], rows_b, gsem).wait()

                @pl.when(cid == 1)
                def _():
                    pltpu.async_copy(tab_hi.at[isl], rows_b, gsem).wait()

                pltpu.async_copy(rows_b, acc.at[idxd2.at[k]], ssem_b,
                                 add=True)

            def pair(k2, c):
                fchunk(k2, 0, rows0, ssem0)
                fchunk(k2, 1, rows1, ssem1)
                return c

            lax.fori_loop(0, _K1_CPT // 2, pair, 0)
            fchunk(_K1_CPT // 2, 0, rows0, ssem0)
            drain_add(rows0, ssem0)
            drain_add(rows1, ssem1)

        plsc.subcore_barrier()
        pltpu.sync_copy(acc.at[pl.ds(zbase, _K1_RPT)],
                        out.at[cid, pl.ds(zbase, _K1_RPT)])
        plsc.subcore_barrier()

    run_pass((ulo, uhi, eu1), ei2, ei2, out_item)
    run_pass((ilo, ihi, ei1), eu2, eu2, out_user)
    run_pass(None, ei2, eu2, out_cnt)


def _k1_call(ulo, uhi, ilo, ihi, eu1, ei1, eu2, ei2, zinit, ones_hbm):
    mesh = plsc.VectorSubcoreMesh(core_axis_name="c", subcore_axis_name="s",
                                  num_cores=NC, num_subcores=NS)
    f = pl.kernel(
        _k1_body,
        out_type=(
            jax.ShapeDtypeStruct((NC, _K1_ACC, DH), jnp.float32),
            jax.ShapeDtypeStruct((NC, _K1_ACC, DH), jnp.float32),
            jax.ShapeDtypeStruct((NC, _K1_ACC, DH), jnp.float32),
        ),
        mesh=mesh,
        scratch_types=[
            pltpu.VMEM_SHARED((_K1_ACC, DH), jnp.float32),
            pltpu.VMEM((_K1_EPT,), jnp.int32),
            pltpu.VMEM((_K1_CPT, _K1_CH), jnp.int32),
            pltpu.VMEM((_K1_CH, DH), jnp.float32),
            pltpu.VMEM((_K1_CH, DH), jnp.float32),
            pltpu.SemaphoreType.DMA,
            pltpu.SemaphoreType.DMA,
            pltpu.SemaphoreType.DMA,
        ],
    )
    return f(ulo, uhi, ilo, ihi, eu1, ei1, eu2, ei2, zinit, ones_hbm)


_K2_R = 2000


def _k2_body(silo, sihi, sulo, suhi, cnti, cntu, wu, bu, wi, bi,
             pu_ref, pi_ref):
    cnt_i = jnp.clip(cnti[0][:, 0:1], 1.0, None)
    h_i = jnp.concatenate([silo[0], sihi[0]], axis=1) / cnt_i
    pi_ref[...] = (jnp.dot(h_i, wi[...], preferred_element_type=jnp.float32)
                   + bi[...])

    cnt_u = jnp.clip(cntu[0][:, 0:1], 1.0, None)
    h_u = jnp.concatenate([sulo[0], suhi[0]], axis=1) / cnt_u
    pu_ref[...] = (jnp.dot(h_u, wu[...], preferred_element_type=jnp.float32)
                   + bu[...])


def _k2_call(sum_item, sum_user, cnt, fc_user_W, fc_user_b, fc_item_W,
             fc_item_b):
    R = _K2_R
    grid = (N_ITEMS // R,)
    return pl.pallas_call(
        _k2_body,
        grid=grid,
        in_specs=[
            pl.BlockSpec((1, R, DH), lambda i: (0, i, 0)),
            pl.BlockSpec((1, R, DH), lambda i: (1, i, 0)),
            pl.BlockSpec((1, R, DH), lambda i: (0, i, 0)),
            pl.BlockSpec((1, R, DH), lambda i: (1, i, 0)),
            pl.BlockSpec((1, R, DH), lambda i: (0, i, 0)),
            pl.BlockSpec((1, R, DH), lambda i: (1, i, 0)),
            pl.BlockSpec((D, D), lambda i: (0, 0)),
            pl.BlockSpec((1, D), lambda i: (0, 0)),
            pl.BlockSpec((D, D), lambda i: (0, 0)),
            pl.BlockSpec((1, D), lambda i: (0, 0)),
        ],
        out_specs=[
            pl.BlockSpec((R, D), lambda i: (i, 0)),
            pl.BlockSpec((R, D), lambda i: (i, 0)),
        ],
        out_shape=[
            jax.ShapeDtypeStruct((N_USERS, D), jnp.float32),
            jax.ShapeDtypeStruct((N_ITEMS, D), jnp.float32),
        ],
    )(sum_item, sum_item, sum_user, sum_user, cnt, cnt, fc_user_W,
      fc_user_b.reshape(1, D), fc_item_W, fc_item_b.reshape(1, D))


_K3_EPT = N_EDGES // (NC * NS)
_K3_CH = 40
_K3_CPT = _K3_EPT // _K3_CH


def _k3_body(uf, pu, itf, pi, eu2, ei2, bias_flat,
             p_all, m_all, q_all, n_all, bias_all,
             idxu2, idxi2,
             rp0, rm0, rq0, rn0, rb0, rp1, rm1, rq1, rn1, rb1,
             gsem, wsem0, wsem1):
    cid = lax.axis_index("c")
    sid = lax.axis_index("s")
    wid = sid * NC + cid
    tbase = wid * _K3_EPT

    pltpu.sync_copy(eu2.at[wid], idxu2)
    pltpu.sync_copy(ei2.at[wid], idxi2)

    def drain(bufs, wsem_b):
        rp, rm, rq, rn, rb = bufs
        base0 = pl.ds(tbase, _K3_CH)
        pltpu.make_async_copy(rp, p_all.at[base0], wsem_b).wait()
        pltpu.make_async_copy(rm, m_all.at[base0], wsem_b).wait()
        pltpu.make_async_copy(rq, q_all.at[base0], wsem_b).wait()
        pltpu.make_async_copy(rn, n_all.at[base0], wsem_b).wait()
        pltpu.make_async_copy(rb, bias_all.at[base0], wsem_b).wait()

    def fchunk(k2, b, bufs, wsem_b):
        k = 2 * k2 + b
        rp, rm, rq, rn, rb = bufs
        base = tbase + k * _K3_CH

        @pl.when(k2 > 0)
        def _():
            drain(bufs, wsem_b)

        iu = idxu2.at[k]
        ii = idxi2.at[k]
        c1 = pltpu.async_copy(uf.at[iu], rp, gsem)
        c2 = pltpu.async_copy(pu.at[iu], rm, gsem)
        c3 = pltpu.async_copy(itf.at[ii], rq, gsem)
        c4 = pltpu.async_copy(pi.at[ii], rn, gsem)
        c5 = pltpu.async_copy(bias_flat.at[ii], rb, gsem)
        c1.wait()
        c2.wait()
        c3.wait()
        c4.wait()
        c5.wait()
        sl = pl.ds(base, _K3_CH)
        pltpu.async_copy(rp, p_all.at[sl], wsem_b)
        pltpu.async_copy(rm, m_all.at[sl], wsem_b)
        pltpu.async_copy(rq, q_all.at[sl], wsem_b)
        pltpu.async_copy(rn, n_all.at[sl], wsem_b)
        pltpu.async_copy(rb, bias_all.at[sl], wsem_b)

    bufs0 = (rp0, rm0, rq0, rn0, rb0)
    bufs1 = (rp1, rm1, rq1, rn1, rb1)

    def pair(k2, c):
        fchunk(k2, 0, bufs0, wsem0)
        fchunk(k2, 1, bufs1, wsem1)
        return c

    lax.fori_loop(0, _K3_CPT // 2, pair, 0)
    fchunk(_K3_CPT // 2, 0, bufs0, wsem0)
    drain(bufs0, wsem0)
    drain(bufs1, wsem1)


def _k3_call(user_feat, prop_user, item_feat, prop_item, eu2, ei2, bias_flat):
    mesh = plsc.VectorSubcoreMesh(core_axis_name="c", subcore_axis_name="s",
                                  num_cores=NC, num_subcores=NS)
    rowbuf = pltpu.VMEM((_K3_CH, D), jnp.float32)
    f = pl.kernel(
        _k3_body,
        out_type=(
            jax.ShapeDtypeStruct((N_EDGES, D), jnp.float32),
            jax.ShapeDtypeStruct((N_EDGES, D), jnp.float32),
            jax.ShapeDtypeStruct((N_EDGES, D), jnp.float32),
            jax.ShapeDtypeStruct((N_EDGES, D), jnp.float32),
            jax.ShapeDtypeStruct((N_EDGES,), jnp.float32),
        ),
        mesh=mesh,
        scratch_types=[
            pltpu.VMEM((_K3_CPT, _K3_CH), jnp.int32),
            pltpu.VMEM((_K3_CPT, _K3_CH), jnp.int32),
            rowbuf, rowbuf, rowbuf, rowbuf,
            pltpu.VMEM((_K3_CH,), jnp.float32),
            rowbuf, rowbuf, rowbuf, rowbuf,
            pltpu.VMEM((_K3_CH,), jnp.float32),
            pltpu.SemaphoreType.DMA,
            pltpu.SemaphoreType.DMA,
            pltpu.SemaphoreType.DMA,
        ],
    )
    return f(user_feat, prop_user, item_feat, prop_item, eu2, ei2, bias_flat)


_K4_E = 1000


def _k4_body(p_ref, m_ref, q_ref, n_ref, bias_ref, w1, b1, w2, b2, w3, b3,
             out_ref):
    p = p_ref[...]
    m = m_ref[...]
    q = q_ref[...]
    n = n_ref[...]
    x = jnp.concatenate([p * q, p * m, n * q, n * m], axis=1)
    y = jnp.dot(x, w1[...], preferred_element_type=jnp.float32) + b1[...]
    y = jnp.maximum(y, 0.0)
    z = jnp.dot(y, w2[...], preferred_element_type=jnp.float32) + b2[...]
    z = jnp.maximum(z, 0.0)
    o = jnp.dot(z, w3[...], preferred_element_type=jnp.float32) + b3[...]
    out_ref[...] = o + bias_ref[...]


def _k4_call(p_all, m_all, q_all, n_all, bias_all, W1, b1, W2, b2, W3, b3):
    E = _K4_E
    grid = (N_EDGES // E,)
    return pl.pallas_call(
        _k4_body,
        grid=grid,
        in_specs=[
            pl.BlockSpec((E, D), lambda i: (i, 0)),
            pl.BlockSpec((E, D), lambda i: (i, 0)),
            pl.BlockSpec((E, D), lambda i: (i, 0)),
            pl.BlockSpec((E, D), lambda i: (i, 0)),
            pl.BlockSpec((E, 1), lambda i: (i, 0)),
            pl.BlockSpec((4 * D, D), lambda i: (0, 0)),
            pl.BlockSpec((1, D), lambda i: (0, 0)),
            pl.BlockSpec((D, 64), lambda i: (0, 0)),
            pl.BlockSpec((1, 64), lambda i: (0, 0)),
            pl.BlockSpec((64, 1), lambda i: (0, 0)),
            pl.BlockSpec((1, 1), lambda i: (0, 0)),
        ],
        out_specs=pl.BlockSpec((E, 1), lambda i: (i, 0)),
        out_shape=jax.ShapeDtypeStruct((N_EDGES, 1), jnp.float32),
    )(p_all, m_all, q_all, n_all, bias_all.reshape(N_EDGES, 1), W1,
      b1.reshape(1, D), W2, b2.reshape(1, 64), W3, b3.reshape(1, 1))


def kernel(user_feat, item_feat, user_bias, item_bias, fc_user_W, fc_user_b,
           fc_item_W, fc_item_b, W1, b1, W2, b2, W3, b3, edge_users,
           edge_items):
    f32 = jnp.float32
    ulo = user_feat[:, :DH]
    uhi = user_feat[:, DH:]
    ilo = item_feat[:, :DH]
    ihi = item_feat[:, DH:]
    zinit = jnp.zeros((_K1_ACC, DH), f32)
    ones80 = jnp.ones((_K1_CH, DH), f32)
    eu2 = edge_users.reshape(NS, _K1_CPT, _K1_CH)
    ei2 = edge_items.reshape(NS, _K1_CPT, _K1_CH)
    eu2b = edge_users.reshape(NC * NS, _K3_CPT, _K3_CH)
    ei2b = edge_items.reshape(NC * NS, _K3_CPT, _K3_CH)
    bias_flat = item_bias.reshape(-1)

    sum_item, sum_user, cnt = _k1_call(ulo, uhi, ilo, ihi, edge_users,
                                       edge_items, eu2, ei2, zinit, ones80)
    prop_user, prop_item = _k2_call(sum_item, sum_user, cnt, fc_user_W,
                                    fc_user_b, fc_item_W, fc_item_b)
    p_all, m_all, q_all, n_all, bias_all = _k3_call(
        user_feat, prop_user, item_feat, prop_item, eu2b, ei2b, bias_flat)
    return _k4_call(p_all, m_all, q_all, n_all, bias_all, W1, b1, W2, b2,
                    W3, b3)


# speedup vs baseline: 2.8735x; 1.0001x over previous
import jax
import jax.numpy as jnp
from jax import lax
from jax.experimental import pallas as pl
from jax.experimental.pallas import tpu as pltpu
from jax.experimental.pallas import tpu_sc as plsc

N_USERS = 10000
N_ITEMS = 10000
N_EDGES = 160000
D = 256
DH = 128

NC, NS = 2, 16


_K1_CH = 80
_K1_EPT = N_EDGES // NS
_K1_CPT = _K1_EPT // _K1_CH
_K1_RPT = 632
_K1_ACC = NS * _K1_RPT


def _k1_body(ulo, uhi, ilo, ihi, eu1, ei1, eu2, ei2, zinit, ones_hbm,
             out_item, out_user, out_cnt,
             acc, idxs1, idxd2, rows0, rows1, gsem, ssem0, ssem1):
    cid = lax.axis_index("c")
    sid = lax.axis_index("s")
    zbase = sid * _K1_RPT

    def drain_add(rows_b, ssem_b):
        pltpu.make_async_copy(rows_b, acc.at[idxd2.at[0]], ssem_b).wait()

    def run_pass(tabs, dst0, dst1, out):
        pltpu.sync_copy(zinit.at[pl.ds(zbase, _K1_RPT)],
                        acc.at[pl.ds(zbase, _K1_RPT)])
        if tabs is None:
            pltpu.sync_copy(ones_hbm, rows0)
        else:
            pltpu.sync_copy(tabs[2].at[pl.ds(sid * _K1_EPT, _K1_EPT)], idxs1)

        @pl.when(cid == 0)
        def _():
            pltpu.sync_copy(dst0.at[sid], idxd2)

        @pl.when(cid == 1)
        def _():
            pltpu.sync_copy(dst1.at[sid], idxd2)

        plsc.subcore_barrier()

        if tabs is None:
            def chunk(k, c):
                @pl.when(k >= 2)
                def _():
                    drain_add(rows0, ssem0)

                pltpu.async_copy(rows0, acc.at[idxd2.at[k]], ssem0, add=True)
                return c

            lax.fori_loop(0, _K1_CPT, chunk, 0)
            drain_add(rows0, ssem0)
            drain_add(rows0, ssem0)
        else:
            tab_lo, tab_hi = tabs[0], tabs[1]

            def fchunk(k2, b, rows_b, ssem_b):
                k = 2 * k2 + b

                @pl.when(k2 > 0)
                def _():
                    drain_add(rows_b, ssem_b)

                isl = idxs1.at[pl.ds(k * _K1_CH, _K1_CH)]

                @pl.when(cid == 0)
                def _():
                    pltpu.async_copy(tab_lo.at[isl], rows_b, gsem).wait()

                @pl.when(cid == 1)
                def _():
                    pltpu.async_copy(tab_hi.at[isl], rows_b, gsem).wait()

                pltpu.async_copy(rows_b, acc.at[idxd2.at[k]], ssem_b,
                                 add=True)

            def pair(k2, c):
                fchunk(k2, 0, rows0, ssem0)
                fchunk(k2, 1, rows1, ssem1)
                return c

            lax.fori_loop(0, _K1_CPT // 2, pair, 0)
            fchunk(_K1_CPT // 2, 0, rows0, ssem0)
            drain_add(rows0, ssem0)
            drain_add(rows1, ssem1)

        plsc.subcore_barrier()
        pltpu.sync_copy(acc.at[pl.ds(zbase, _K1_RPT)],
                        out.at[cid, pl.ds(zbase, _K1_RPT)])
        plsc.subcore_barrier()

    run_pass((ulo, uhi, eu1), ei2, ei2, out_item)
    run_pass((ilo, ihi, ei1), eu2, eu2, out_user)
    run_pass(None, ei2, eu2, out_cnt)


def _k1_call(ulo, uhi, ilo, ihi, eu1, ei1, eu2, ei2, zinit, ones_hbm):
    mesh = plsc.VectorSubcoreMesh(core_axis_name="c", subcore_axis_name="s",
                                  num_cores=NC, num_subcores=NS)
    f = pl.kernel(
        _k1_body,
        out_type=(
            jax.ShapeDtypeStruct((NC, _K1_ACC, DH), jnp.float32),
            jax.ShapeDtypeStruct((NC, _K1_ACC, DH), jnp.float32),
            jax.ShapeDtypeStruct((NC, _K1_ACC, DH), jnp.float32),
        ),
        mesh=mesh,
        scratch_types=[
            pltpu.VMEM_SHARED((_K1_ACC, DH), jnp.float32),
            pltpu.VMEM((_K1_EPT,), jnp.int32),
            pltpu.VMEM((_K1_CPT, _K1_CH), jnp.int32),
            pltpu.VMEM((_K1_CH, DH), jnp.float32),
            pltpu.VMEM((_K1_CH, DH), jnp.float32),
            pltpu.SemaphoreType.DMA,
            pltpu.SemaphoreType.DMA,
            pltpu.SemaphoreType.DMA,
        ],
    )
    return f(ulo, uhi, ilo, ihi, eu1, ei1, eu2, ei2, zinit, ones_hbm)


_K2_R = 2000


def _k2_body(silo, sihi, sulo, suhi, cnti, cntu, wu, bu, wi, bi,
             pu_ref, pi_ref):
    cnt_i = jnp.clip(cnti[0][:, 0:1], 1.0, None)
    h_i = jnp.concatenate([silo[0], sihi[0]], axis=1) / cnt_i
    pi_ref[...] = (jnp.dot(h_i, wi[...], preferred_element_type=jnp.float32)
                   + bi[...])

    cnt_u = jnp.clip(cntu[0][:, 0:1], 1.0, None)
    h_u = jnp.concatenate([sulo[0], suhi[0]], axis=1) / cnt_u
    pu_ref[...] = (jnp.dot(h_u, wu[...], preferred_element_type=jnp.float32)
                   + bu[...])


def _k2_call(sum_item, sum_user, cnt, fc_user_W, fc_user_b, fc_item_W,
             fc_item_b):
    R = _K2_R
    grid = (N_ITEMS // R,)
    return pl.pallas_call(
        _k2_body,
        grid=grid,
        in_specs=[
            pl.BlockSpec((1, R, DH), lambda i: (0, i, 0)),
            pl.BlockSpec((1, R, DH), lambda i: (1, i, 0)),
            pl.BlockSpec((1, R, DH), lambda i: (0, i, 0)),
            pl.BlockSpec((1, R, DH), lambda i: (1, i, 0)),
            pl.BlockSpec((1, R, DH), lambda i: (0, i, 0)),
            pl.BlockSpec((1, R, DH), lambda i: (1, i, 0)),
            pl.BlockSpec((D, D), lambda i: (0, 0)),
            pl.BlockSpec((1, D), lambda i: (0, 0)),
            pl.BlockSpec((D, D), lambda i: (0, 0)),
            pl.BlockSpec((1, D), lambda i: (0, 0)),
        ],
        out_specs=[
            pl.BlockSpec((R, D), lambda i: (i, 0)),
            pl.BlockSpec((R, D), lambda i: (i, 0)),
        ],
        out_shape=[
            jax.ShapeDtypeStruct((N_USERS, D), jnp.float32),
            jax.ShapeDtypeStruct((N_ITEMS, D), jnp.float32),
        ],
    )(sum_item, sum_item, sum_user, sum_user, cnt, cnt, fc_user_W,
      fc_user_b.reshape(1, D), fc_item_W, fc_item_b.reshape(1, D))


_K3_EPT = N_EDGES // (NC * NS)
_K3_CH = 40
_K3_CPT = _K3_EPT // _K3_CH


def _k3_body(uf, pu, itf, pi, eu2, ei2, bias_flat,
             p_all, m_all, q_all, n_all, bias_all,
             idxu2, idxi2,
             rp0, rm0, rq0, rn0, rb0, rp1, rm1, rq1, rn1, rb1,
             gsem, wsem0, wsem1):
    cid = lax.axis_index("c")
    sid = lax.axis_index("s")
    wid = sid * NC + cid
    tbase = wid * _K3_EPT

    pltpu.sync_copy(eu2.at[wid], idxu2)
    pltpu.sync_copy(ei2.at[wid], idxi2)

    def drain(bufs, wsem_b):
        rp, rm, rq, rn, rb = bufs
        base0 = pl.ds(tbase, _K3_CH)
        pltpu.make_async_copy(rp, p_all.at[base0], wsem_b).wait()
        pltpu.make_async_copy(rm, m_all.at[base0], wsem_b).wait()
        pltpu.make_async_copy(rq, q_all.at[base0], wsem_b).wait()
        pltpu.make_async_copy(rn, n_all.at[base0], wsem_b).wait()
        pltpu.make_async_copy(rb, bias_all.at[base0], wsem_b).wait()

    def fchunk(k2, b, bufs, wsem_b):
        k = 2 * k2 + b
        rp, rm, rq, rn, rb = bufs
        base = tbase + k * _K3_CH

        @pl.when(k2 > 0)
        def _():
            drain(bufs, wsem_b)

        iu = idxu2.at[k]
        ii = idxi2.at[k]
        c1 = pltpu.async_copy(uf.at[iu], rp, gsem)
        c2 = pltpu.async_copy(pu.at[iu], rm, gsem)
        c3 = pltpu.async_copy(itf.at[ii], rq, gsem)
        c4 = pltpu.async_copy(pi.at[ii], rn, gsem)
        c5 = pltpu.async_copy(bias_flat.at[ii], rb, gsem)
        c1.wait()
        c2.wait()
        c3.wait()
        c4.wait()
        c5.wait()
        sl = pl.ds(base, _K3_CH)
        pltpu.async_copy(rp, p_all.at[sl], wsem_b)
        pltpu.async_copy(rm, m_all.at[sl], wsem_b)
        pltpu.async_copy(rq, q_all.at[sl], wsem_b)
        pltpu.async_copy(rn, n_all.at[sl], wsem_b)
        pltpu.async_copy(rb, bias_all.at[sl], wsem_b)

    bufs0 = (rp0, rm0, rq0, rn0, rb0)
    bufs1 = (rp1, rm1, rq1, rn1, rb1)

    def pair(k2, c):
        fchunk(k2, 0, bufs0, wsem0)
        fchunk(k2, 1, bufs1, wsem1)
        return c

    lax.fori_loop(0, _K3_CPT // 2, pair, 0)
    fchunk(_K3_CPT // 2, 0, bufs0, wsem0)
    drain(bufs0, wsem0)
    drain(bufs1, wsem1)


def _k3_call(user_feat, prop_user, item_feat, prop_item, eu2, ei2, bias_flat):
    mesh = plsc.VectorSubcoreMesh(core_axis_name="c", subcore_axis_name="s",
                                  num_cores=NC, num_subcores=NS)
    rowbuf = pltpu.VMEM((_K3_CH, D), jnp.float32)
    f = pl.kernel(
        _k3_body,
        out_type=(
            jax.ShapeDtypeStruct((N_EDGES, D), jnp.float32),
            jax.ShapeDtypeStruct((N_EDGES, D), jnp.float32),
            jax.ShapeDtypeStruct((N_EDGES, D), jnp.float32),
            jax.ShapeDtypeStruct((N_EDGES, D), jnp.float32),
            jax.ShapeDtypeStruct((N_EDGES,), jnp.float32),
        ),
        mesh=mesh,
        scratch_types=[
            pltpu.VMEM((_K3_CPT, _K3_CH), jnp.int32),
            pltpu.VMEM((_K3_CPT, _K3_CH), jnp.int32),
            rowbuf, rowbuf, rowbuf, rowbuf,
            pltpu.VMEM((_K3_CH,), jnp.float32),
            rowbuf, rowbuf, rowbuf, rowbuf,
            pltpu.VMEM((_K3_CH,), jnp.float32),
            pltpu.SemaphoreType.DMA,
            pltpu.SemaphoreType.DMA,
            pltpu.SemaphoreType.DMA,
        ],
    )
    return f(user_feat, prop_user, item_feat, prop_item, eu2, ei2, bias_flat)


_K4_E = 1000


def _k4_body(p_ref, m_ref, q_ref, n_ref, bias_ref, w1, b1, w2, b2, w3, b3,
             out_ref):
    p = p_ref[...]
    m = m_ref[...]
    q = q_ref[...]
    n = n_ref[...]
    x = jnp.concatenate([p * q, p * m, n * q, n * m],
                        axis=1).astype(jnp.bfloat16)
    y = jnp.dot(x, w1[...], preferred_element_type=jnp.float32) + b1[...]
    y = jnp.maximum(y, 0.0).astype(jnp.bfloat16)
    z = jnp.dot(y, w2[...], preferred_element_type=jnp.float32) + b2[...]
    z = jnp.maximum(z, 0.0)
    o = jnp.dot(z, w3[...], preferred_element_type=jnp.float32) + b3[...]
    out_ref[...] = o + bias_ref[...]


def _k4_call(p_all, m_all, q_all, n_all, bias_all, W1, b1, W2, b2, W3, b3):
    E = _K4_E
    grid = (N_EDGES // E,)
    return pl.pallas_call(
        _k4_body,
        grid=grid,
        in_specs=[
            pl.BlockSpec((E, D), lambda i: (i, 0)),
            pl.BlockSpec((E, D), lambda i: (i, 0)),
            pl.BlockSpec((E, D), lambda i: (i, 0)),
            pl.BlockSpec((E, D), lambda i: (i, 0)),
            pl.BlockSpec((E, 1), lambda i: (i, 0)),
            pl.BlockSpec((4 * D, D), lambda i: (0, 0)),
            pl.BlockSpec((1, D), lambda i: (0, 0)),
            pl.BlockSpec((D, 64), lambda i: (0, 0)),
            pl.BlockSpec((1, 64), lambda i: (0, 0)),
            pl.BlockSpec((64, 1), lambda i: (0, 0)),
            pl.BlockSpec((1, 1), lambda i: (0, 0)),
        ],
        out_specs=pl.BlockSpec((E, 1), lambda i: (i, 0)),
        out_shape=jax.ShapeDtypeStruct((N_EDGES, 1), jnp.float32),
    )(p_all, m_all, q_all, n_all, bias_all.reshape(N_EDGES, 1),
      W1.astype(jnp.bfloat16), b1.reshape(1, D),
      W2.astype(jnp.bfloat16), b2.reshape(1, 64), W3, b3.reshape(1, 1))


def kernel(user_feat, item_feat, user_bias, item_bias, fc_user_W, fc_user_b,
           fc_item_W, fc_item_b, W1, b1, W2, b2, W3, b3, edge_users,
           edge_items):
    f32 = jnp.float32
    ulo = user_feat[:, :DH]
    uhi = user_feat[:, DH:]
    ilo = item_feat[:, :DH]
    ihi = item_feat[:, DH:]
    zinit = jnp.zeros((_K1_ACC, DH), f32)
    ones80 = jnp.ones((_K1_CH, DH), f32)
    eu2 = edge_users.reshape(NS, _K1_CPT, _K1_CH)
    ei2 = edge_items.reshape(NS, _K1_CPT, _K1_CH)
    eu2b = edge_users.reshape(NC * NS, _K3_CPT, _K3_CH)
    ei2b = edge_items.reshape(NC * NS, _K3_CPT, _K3_CH)
    bias_flat = item_bias.reshape(-1)

    sum_item, sum_user, cnt = _k1_call(ulo, uhi, ilo, ihi, edge_users,
                                       edge_items, eu2, ei2, zinit, ones80)
    prop_user, prop_item = _k2_call(sum_item, sum_user, cnt, fc_user_W,
                                    fc_user_b, fc_item_W, fc_item_b)
    p_all, m_all, q_all, n_all, bias_all = _k3_call(
        user_feat, prop_user, item_feat, prop_item, eu2b, ei2b, bias_flat)
    return _k4_call(p_all, m_all, q_all, n_all, bias_all, W1, b1, W2, b2,
                    W3, b3)
